# Initial kernel scaffold; baseline (speedup 1.0000x reference)
import jax
import jax.numpy as jnp
from jax.experimental import pallas as pl


def kernel(x, latent, gW1, gb1, gW2, gb2, gln_g, gln_b, gW3, gb3, eW0, eb0, eWh, ebh, eWs, ebs, eWo, ebo):
    raise NotImplementedError("write your pallas kernel here")


# speedup vs baseline: 2.4863x; 2.4863x over previous
import functools

import jax
import jax.numpy as jnp
import numpy as np
from jax.experimental import pallas as pl
from jax.experimental.pallas import tpu as pltpu

E = 8
K = 4
D_ENC = 256
D_LAT = 256
HID = 256
OUT = 4
DEPTH = 7
SKIP = 5
CH = D_LAT // E
EIN = D_ENC + CH
GIN = D_ENC + D_LAT

TB = 512


def _gate_topk(xb, lb, gW1, gb1, gW2, gb2, gln_g, gln_b, gW3, gb3):
    dot = functools.partial(jnp.dot, preferred_element_type=jnp.float32)
    g = jax.nn.relu(dot(xb, gW1[:D_ENC]) + dot(lb, gW1[D_ENC:]) + gb1)
    g = dot(g, gW2) + gb2
    m = g.mean(-1, keepdims=True)
    v = ((g - m) ** 2).mean(-1, keepdims=True)
    g = (g - m) / jnp.sqrt(v + 1e-5) * gln_g + gln_b
    logits = dot(g, gW3) + gb3
    mx = logits.max(-1, keepdims=True)
    s = jnp.exp(logits - mx)
    s = s / s.sum(-1, keepdims=True)
    lanes = jax.lax.broadcasted_iota(jnp.int32, s.shape, 1)
    cur = s
    sum4 = jnp.zeros(s.shape[:1] + (1,), jnp.float32)
    eidx = None
    gmax = None
    for r in range(K):
        m_r = cur.max(-1, keepdims=True)
        i_r = jnp.where(cur == m_r, lanes, E).min(-1, keepdims=True)
        sum4 = sum4 + m_r
        if r == 0:
            eidx = i_r
            gmax = m_r
        cur = jnp.where(lanes == i_r, -jnp.inf, cur)
    gval = gmax / (sum4 + 1e-9)
    return eidx, gval


def _expert_mlp(h0, W0, b0, Wh, bh, Ws, bs, Wo, bo):
    dot = functools.partial(jnp.dot, preferred_element_type=jnp.float32)
    h = jax.nn.relu(dot(h0, W0) + b0)
    hidx = 0
    for i in range(1, DEPTH):
        if i == SKIP:
            h = jnp.concatenate([h, h0], axis=-1)
            h = jax.nn.relu(dot(h, Ws) + bs)
        else:
            h = jax.nn.relu(dot(h, Wh[hidx]) + bh[hidx])
            hidx += 1
    return dot(h, Wo) + bo


def _dense_body(x_ref, lat_ref, gW1, gb1, gW2, gb2, gln_g, gln_b, gW3, gb3,
                eW0, eb0, eWh, ebh, eWs, ebs, eWo, ebo, out_ref):
    xb = x_ref[...]
    lb = lat_ref[...]
    eidx, gval = _gate_topk(xb, lb, gW1[...], gb1[...], gW2[...], gb2[...],
                            gln_g[...], gln_b[...], gW3[...], gb3[...])
    sel = jnp.zeros((xb.shape[0], OUT), jnp.float32)
    for e in range(E):
        h0 = jnp.concatenate([xb, lb[:, e * CH:(e + 1) * CH]], axis=-1)
        o = _expert_mlp(h0, eW0[e], eb0[e], eWh[e], ebh[e], eWs[e], ebs[e],
                        eWo[e], ebo[e])
        sel = jnp.where(eidx == e, o, sel)
    c = jnp.exp(sel) * gval
    c = jnp.where(c == 0, jnp.float32(np.finfo(np.float32).eps), c)
    out_ref[...] = jnp.log(c)


def kernel(x, latent, gW1, gb1, gW2, gb2, gln_g, gln_b, gW3, gb3,
           eW0, eb0, eWh, ebh, eWs, ebs, eWo, ebo):
    T = x.shape[0]
    grid = (T // TB,)
    tok = lambda i: (i, 0)
    whole = pl.BlockSpec(lambda i: tuple([0] * 1))
    def wspec(a):
        return pl.BlockSpec(a.shape, lambda i: tuple([0] * a.ndim))
    out = pl.pallas_call(
        _dense_body,
        grid=grid,
        in_specs=[
            pl.BlockSpec((TB, D_ENC), tok),
            pl.BlockSpec((TB, D_LAT), tok),
        ] + [wspec(a) for a in (gW1, gb1, gW2, gb2, gln_g, gln_b, gW3, gb3,
                                eW0, eb0, eWh, ebh, eWs, ebs, eWo, ebo)],
        out_specs=pl.BlockSpec((TB, OUT), tok),
        out_shape=jax.ShapeDtypeStruct((T, OUT), jnp.float32),
    )(x, latent, gW1, gb1, gW2, gb2, gln_g, gln_b, gW3, gb3,
      eW0, eb0, eWh, ebh, eWs, ebs, eWo, ebo)
    return out
